# baseline (speedup 1.0000x reference)
import jax
import jax.numpy as jnp
from jax import lax
from jax.experimental import pallas as pl
from jax.experimental.pallas import tpu as pltpu

_B, _T, _NF = 16, 1024, 198
_CD = 256
_NCB = 512
_SH = _NF // 2
_TQ = _T // 4


def _sd(x):
    return jnp.concatenate([jnp.zeros_like(x[:1]), x[:-1]], axis=0)


def _su(x):
    return jnp.concatenate([x[1:], jnp.zeros_like(x[:1])], axis=0)


def _mm(a, b):
    return jnp.dot(a, b, preferred_element_type=jnp.float32)


def _act(x):
    return jnp.maximum(x, 0.0).astype(jnp.bfloat16)


def _path(x4, w0, wl1, wl2, wl3, wd0, wd1, wd2, wd3, cbT, cb, cb2, hp):
    cat = lambda *ps: jnp.concatenate(ps, axis=1)

    xs, xu = _sd(x4), _su(x4)
    if hp:
        h0 = _act(_mm(xs, w0[0]) + _mm(xs, w0[6])
                  + _mm(x4, w0[1]) + _mm(x4, w0[7]))
        h1 = _act(_mm(x4, w0[2]) + _mm(x4, w0[8]))
        h2 = _act(_mm(x4, w0[3]) + _mm(x4, w0[9]))
        h3 = _act(_mm(x4, w0[4]) + _mm(x4, w0[10])
                  + _mm(xu, w0[5]) + _mm(xu, w0[11]))
    else:
        h0 = _act(_mm(xs, w0[0]) + _mm(x4, w0[1]))
        h1 = _act(_mm(x4, w0[2]))
        h2 = _act(_mm(x4, w0[3]))
        h3 = _act(_mm(x4, w0[4]) + _mm(_su(x4), w0[5]))
    ye = _act(_mm(cat(_sd(h3), h0, h1, h2), wl1))
    yo = _act(_mm(cat(h1, h2, h3, _su(h0)), wl1))
    h = _act(_mm(cat(_sd(yo), ye, yo, _su(ye)), wl2))
    hs, hu = _sd(h), _su(h)
    z = _mm(cat(hs, hs, h, h, hu, hu), wl3)

    zb = z.astype(jnp.bfloat16)
    zz = jnp.sum(z * z, axis=-1, keepdims=True)
    zc = _mm(cat(zb, zb), cbT)
    d = zz - 2.0 * zc + cb2
    idx = jnp.argmin(d, axis=-1)[:, None]
    oh = (lax.broadcasted_iota(jnp.int32, (_TQ, _NCB), 1) == idx
          ).astype(jnp.bfloat16)
    q = _mm(oh, cb).astype(jnp.bfloat16)

    qs, qu = _sd(q), _su(q)
    h = _act(_mm(cat(qs, qs, q, q, qu, qu), wd0))
    hs, hu = _sd(h), _su(h)
    ge = _act(_mm(cat(hs, hs, h, h, h, h), wd1))
    go = _act(_mm(cat(h, h, h, h, hu, hu), wd1))
    gos, geu = _sd(go), _su(ge)
    o0 = _act(_mm(cat(gos, gos, ge, ge, ge, ge), wd2))
    o1 = _act(_mm(cat(ge, ge, ge, ge, go, go), wd2))
    o2 = _act(_mm(cat(ge, ge, go, go, go, go), wd2))
    o3 = _act(_mm(cat(go, go, go, go, geu, geu), wd2))

    o3s, o0u = _sd(o3), _su(o0)
    if hp:
        y0 = _mm(cat(o3s, o3s, o0, o0, o1, o1), wd3)
        y1 = _mm(cat(o0, o0, o1, o1, o2, o2), wd3)
        y2 = _mm(cat(o1, o1, o2, o2, o3, o3), wd3)
        y3 = _mm(cat(o2, o2, o3, o3, o0u, o0u), wd3)
    else:
        y0 = _mm(cat(o3s, o0, o1), wd3)
        y1 = _mm(cat(o0, o1, o2), wd3)
        y2 = _mm(cat(o1, o2, o3), wd3)
        y3 = _mm(cat(o2, o3, o0u), wd3)
    return y0, y1, y2, y3


def _body(tin, hin,
          tw0, tl1, tl2, tl3, td0, td1, td2, td3, tcbT, tcb, tcb2,
          hw0, hl1, hl2, hl3, hd0, hd1, hd2, hd3, hcbT, hcb, hcb2,
          t0, t1, t2, t3, p0, p1, p2, p3):
    ty = _path(tin[0], tw0, tl1[...], tl2[...], tl3[...], td0[...],
               td1[...], td2[...], td3[...], tcbT[...], tcb[...],
               tcb2[...], False)
    t0[0], t1[0], t2[0], t3[0] = ty
    hy = _path(hin[0], hw0, hl1[...], hl2[...], hl3[...], hd0[...],
               hd1[...], hd2[...], hd3[...], hcbT[...], hcb[...],
               hcb2[...], True)
    p0[0], p1[0], p2[0], p3[0] = hy


def _taps(w):
    return [w[:, :, k].T for k in range(w.shape[2])]


def _lo(m):
    return (m - m.astype(jnp.bfloat16).astype(jnp.float32)).astype(
        jnp.bfloat16).astype(jnp.float32)


def _kcat(mats):
    return jnp.concatenate(mats, axis=0).astype(jnp.bfloat16)


def _khilo(w):
    t = _taps(w)
    out = []
    for m in t:
        out += [m, _lo(m)]
    return _kcat(out)


def _pack_l0(w, with_lo):
    t0, t1, t2 = _taps(w)
    z = jnp.zeros_like(t0)

    def blk(b0, b1, b2, b3):
        return jnp.concatenate([b0, b1, b2, b3], axis=0)

    mats = [
        blk(z, z, z, t0),
        blk(t1, t2, z, z),
        blk(t0, t1, t2, z),
        blk(z, t0, t1, t2),
        blk(z, z, t0, t1),
        blk(t2, z, z, z),
    ]
    if with_lo:
        mats += [_lo(m) for m in mats]
    return jnp.stack(mats).astype(jnp.bfloat16)


def kernel(features, traj_enc_w0, traj_enc_w1, traj_enc_w2, traj_enc_w3,
           traj_codebook, traj_dec_w0, traj_dec_w1, traj_dec_w2, traj_dec_w3,
           hp_enc_w0, hp_enc_w1, hp_enc_w2, hp_enc_w3, hp_codebook,
           hp_dec_w0, hp_dec_w1, hp_dec_w2, hp_dec_w3):
    Bs = features.shape[0]
    n = 2 * Bs
    x = jnp.concatenate([features[:, :, :_SH], features[:, :, _SH:]], axis=0)
    tin = jnp.concatenate([x[..., :6], x[..., _SH - 3:]], axis=-1)
    hin = x[..., 6:_SH - 3]
    tin4 = tin.reshape(n, _TQ, 4 * 9).astype(jnp.bfloat16)
    hin4 = hin.reshape(n, _TQ, 4 * 90).astype(jnp.bfloat16)

    tw0 = _pack_l0(traj_enc_w0, False)
    hw0 = _pack_l0(hp_enc_w0, True)
    tl1, tl2 = _kcat(_taps(traj_enc_w1)), _kcat(_taps(traj_enc_w2))
    hl1, hl2 = _kcat(_taps(hp_enc_w1)), _kcat(_taps(hp_enc_w2))
    tl3, td0 = _khilo(traj_enc_w3), _khilo(traj_dec_w0)
    td1, td2 = _khilo(traj_dec_w1), _khilo(traj_dec_w2)
    hl3, hd0 = _khilo(hp_enc_w3), _khilo(hp_dec_w0)
    hd1, hd2 = _khilo(hp_dec_w1), _khilo(hp_dec_w2)
    td3 = _kcat(_taps(traj_dec_w3))
    hd3 = _khilo(hp_dec_w3)
    tcbT = _kcat([traj_codebook.T, _lo(traj_codebook.T)])
    hcbT = _kcat([hp_codebook.T, _lo(hp_codebook.T)])
    tcb = traj_codebook.astype(jnp.bfloat16)
    hcb = hp_codebook.astype(jnp.bfloat16)
    tcb2 = jnp.sum(traj_codebook * traj_codebook, -1)[None]
    hcb2 = jnp.sum(hp_codebook * hp_codebook, -1)[None]

    full = lambda a: pl.BlockSpec(a.shape, lambda i: (0,) * a.ndim)
    item = lambda c: pl.BlockSpec((1, _TQ, c), lambda i: (i, 0, 0))
    oph = lambda c: jax.ShapeDtypeStruct((n, _TQ, c), jnp.float32)

    tws = [tw0, tl1, tl2, tl3, td0, td1, td2, td3, tcbT, tcb, tcb2]
    hws = [hw0, hl1, hl2, hl3, hd0, hd1, hd2, hd3, hcbT, hcb, hcb2]
    outs = pl.pallas_call(
        _body,
        grid=(n,),
        in_specs=[item(36), item(360)]
                 + [full(a) for a in tws] + [full(a) for a in hws],
        out_specs=[item(9)] * 4 + [item(90)] * 4,
        out_shape=[oph(9)] * 4 + [oph(90)] * 4,
        compiler_params=pltpu.CompilerParams(
            dimension_semantics=("parallel",)),
    )(tin4, hin4, *tws, *hws)

    tout = jnp.stack(outs[0:4], axis=2).reshape(n, _T, 9)
    hout = jnp.stack(outs[4:8], axis=2).reshape(n, _T, 90)
    xo = jnp.concatenate([tout[..., :6], hout, tout[..., 6:]], axis=-1)
    x_out = jnp.concatenate([xo[:Bs], xo[Bs:]], axis=-1)
    return (x_out, jnp.array([1e30], jnp.float32),
            jnp.array([1e30], jnp.float32))


# speedup vs baseline: 1.0750x; 1.0750x over previous
import jax
import jax.numpy as jnp
from jax import lax
from jax.experimental import pallas as pl
from jax.experimental.pallas import tpu as pltpu

_B, _T, _NF = 16, 1024, 198
_CD = 256
_NCB = 512
_SH = _NF // 2
_TQ = _T // 4


def _sd(x):
    return jnp.concatenate([jnp.zeros_like(x[:1]), x[:-1]], axis=0)


def _su(x):
    return jnp.concatenate([x[1:], jnp.zeros_like(x[:1])], axis=0)


def _mm(a, b):
    return jnp.dot(a, b, preferred_element_type=jnp.float32)


def _act(x):
    return jnp.maximum(x, 0.0).astype(jnp.bfloat16)


def _hl(a, w, n=_CD):
    r = jnp.dot(a, w, preferred_element_type=jnp.float32)
    return r[:, :n] + r[:, n:2 * n]


def _hl90(a, w):
    r = jnp.dot(a, w, preferred_element_type=jnp.float32)
    return r[:, 0:90] + r[:, 128:218]


def _path(x4, w0, wl1, wl2, wl3, wd0, wd1, wd2, wd3, cbT, cb, cb2, hp):
    cat = lambda *ps: jnp.concatenate(ps, axis=1)

    xs, xu = _sd(x4), _su(x4)
    if hp:
        h0 = _act(_hl(xs, w0[0]) + _hl(x4, w0[1]))
        h1 = _act(_hl(x4, w0[2]))
        h2 = _act(_hl(x4, w0[3]))
        h3 = _act(_hl(x4, w0[4]) + _hl(xu, w0[5]))
    else:
        h0 = _act(_mm(xs, w0[0]) + _mm(x4, w0[1]))
        h1 = _act(_mm(x4, w0[2]))
        h2 = _act(_mm(x4, w0[3]))
        h3 = _act(_mm(x4, w0[4]) + _mm(_su(x4), w0[5]))
    ye = _act(_mm(cat(_sd(h3), h0, h1, h2), wl1))
    yo = _act(_mm(cat(h1, h2, h3, _su(h0)), wl1))
    h = _act(_mm(cat(_sd(yo), ye, yo, _su(ye)), wl2))
    hs, hu = _sd(h), _su(h)
    z = _hl(hs, wl3[0]) + _hl(h, wl3[1]) + _hl(hu, wl3[2])

    zb = z.astype(jnp.bfloat16)
    zz = jnp.sum(z * z, axis=-1, keepdims=True)
    zc = _hl(zb, cbT, n=_NCB)
    d = zz - 2.0 * zc + cb2
    idx = jnp.argmin(d, axis=-1)[:, None]
    oh = (lax.broadcasted_iota(jnp.int32, (_TQ, _NCB), 1) == idx
          ).astype(jnp.bfloat16)
    q = _mm(oh, cb).astype(jnp.bfloat16)

    qs, qu = _sd(q), _su(q)
    h = _act(_hl(qs, wd0[0]) + _hl(q, wd0[1]) + _hl(qu, wd0[2]))
    hs, hu = _sd(h), _su(h)
    ge = _act(_hl(hs, wd1[0]) + _hl(h, wd1[1]) + _hl(h, wd1[2]))
    go = _act(_hl(h, wd1[0]) + _hl(h, wd1[1]) + _hl(hu, wd1[2]))
    gos, geu = _sd(go), _su(ge)
    o0 = _act(_hl(gos, wd2[0]) + _hl(ge, wd2[1]) + _hl(ge, wd2[2]))
    o1 = _act(_hl(ge, wd2[0]) + _hl(ge, wd2[1]) + _hl(go, wd2[2]))
    o2 = _act(_hl(ge, wd2[0]) + _hl(go, wd2[1]) + _hl(go, wd2[2]))
    o3 = _act(_hl(go, wd2[0]) + _hl(go, wd2[1]) + _hl(geu, wd2[2]))

    o3s, o0u = _sd(o3), _su(o0)
    if hp:
        y0 = _hl90(o3s, wd3[0]) + _hl90(o0, wd3[1]) + _hl90(o1, wd3[2])
        y1 = _hl90(o0, wd3[0]) + _hl90(o1, wd3[1]) + _hl90(o2, wd3[2])
        y2 = _hl90(o1, wd3[0]) + _hl90(o2, wd3[1]) + _hl90(o3, wd3[2])
        y3 = _hl90(o2, wd3[0]) + _hl90(o3, wd3[1]) + _hl90(o0u, wd3[2])
    else:
        y0 = _mm(cat(o3s, o0, o1), wd3)
        y1 = _mm(cat(o0, o1, o2), wd3)
        y2 = _mm(cat(o1, o2, o3), wd3)
        y3 = _mm(cat(o2, o3, o0u), wd3)
    return y0, y1, y2, y3


def _body(tin, hin,
          tw0, tl1, tl2, tl3, td0, td1, td2, td3, tcbT, tcb, tcb2,
          hw0, hl1, hl2, hl3, hd0, hd1, hd2, hd3, hcbT, hcb, hcb2,
          t0, t1, t2, t3, p0, p1, p2, p3):
    ty = _path(tin[0], tw0, tl1[...], tl2[...], tl3[...], td0[...],
               td1[...], td2[...], td3[...], tcbT[...], tcb[...],
               tcb2[...], False)
    t0[0], t1[0], t2[0], t3[0] = ty
    hy = _path(hin[0], hw0, hl1[...], hl2[...], hl3[...], hd0[...],
               hd1[...], hd2[...], hd3[...], hcbT[...], hcb[...],
               hcb2[...], True)
    p0[0], p1[0], p2[0], p3[0] = hy


def _taps(w):
    return [w[:, :, k].T for k in range(w.shape[2])]


def _lo(m):
    return (m - m.astype(jnp.bfloat16).astype(jnp.float32)).astype(
        jnp.bfloat16).astype(jnp.float32)


def _kcat(mats):
    return jnp.concatenate(mats, axis=0).astype(jnp.bfloat16)


def _npair(m, pad=0):
    z = jnp.zeros((m.shape[0], pad), m.dtype)
    return jnp.concatenate([m, z, _lo(m), z], axis=1).astype(jnp.bfloat16)


def _nhilo(w, pad=0):
    return jnp.stack([_npair(m, pad) for m in _taps(w)])


def _pack_l0(w, with_lo):
    t0, t1, t2 = _taps(w)
    z = jnp.zeros_like(t0)

    def blk(b0, b1, b2, b3):
        return jnp.concatenate([b0, b1, b2, b3], axis=0)

    mats = [
        blk(z, z, z, t0),
        blk(t1, t2, z, z),
        blk(t0, t1, t2, z),
        blk(z, t0, t1, t2),
        blk(z, z, t0, t1),
        blk(t2, z, z, z),
    ]
    if with_lo:
        mats = [jnp.concatenate([m, _lo(m)], axis=1) for m in mats]
    return jnp.stack(mats).astype(jnp.bfloat16)


def kernel(features, traj_enc_w0, traj_enc_w1, traj_enc_w2, traj_enc_w3,
           traj_codebook, traj_dec_w0, traj_dec_w1, traj_dec_w2, traj_dec_w3,
           hp_enc_w0, hp_enc_w1, hp_enc_w2, hp_enc_w3, hp_codebook,
           hp_dec_w0, hp_dec_w1, hp_dec_w2, hp_dec_w3):
    Bs = features.shape[0]
    n = 2 * Bs
    x = jnp.concatenate([features[:, :, :_SH], features[:, :, _SH:]], axis=0)
    tin = jnp.concatenate([x[..., :6], x[..., _SH - 3:]], axis=-1)
    hin = x[..., 6:_SH - 3]
    tin4 = tin.reshape(n, _TQ, 4 * 9).astype(jnp.bfloat16)
    hin4 = hin.reshape(n, _TQ, 4 * 90).astype(jnp.bfloat16)

    tw0 = _pack_l0(traj_enc_w0, False)
    hw0 = _pack_l0(hp_enc_w0, True)
    tl1, tl2 = _kcat(_taps(traj_enc_w1)), _kcat(_taps(traj_enc_w2))
    hl1, hl2 = _kcat(_taps(hp_enc_w1)), _kcat(_taps(hp_enc_w2))
    tl3, td0 = _nhilo(traj_enc_w3), _nhilo(traj_dec_w0)
    td1, td2 = _nhilo(traj_dec_w1), _nhilo(traj_dec_w2)
    hl3, hd0 = _nhilo(hp_enc_w3), _nhilo(hp_dec_w0)
    hd1, hd2 = _nhilo(hp_dec_w1), _nhilo(hp_dec_w2)
    td3 = _kcat(_taps(traj_dec_w3))
    hd3 = _nhilo(hp_dec_w3, pad=38)
    tcbT = jnp.concatenate([traj_codebook.T, _lo(traj_codebook.T)],
                           axis=1).astype(jnp.bfloat16)
    hcbT = jnp.concatenate([hp_codebook.T, _lo(hp_codebook.T)],
                           axis=1).astype(jnp.bfloat16)
    tcb = traj_codebook.astype(jnp.bfloat16)
    hcb = hp_codebook.astype(jnp.bfloat16)
    tcb2 = jnp.sum(traj_codebook * traj_codebook, -1)[None]
    hcb2 = jnp.sum(hp_codebook * hp_codebook, -1)[None]

    full = lambda a: pl.BlockSpec(a.shape, lambda i: (0,) * a.ndim)
    item = lambda c: pl.BlockSpec((1, _TQ, c), lambda i: (i, 0, 0))
    oph = lambda c: jax.ShapeDtypeStruct((n, _TQ, c), jnp.float32)

    tws = [tw0, tl1, tl2, tl3, td0, td1, td2, td3, tcbT, tcb, tcb2]
    hws = [hw0, hl1, hl2, hl3, hd0, hd1, hd2, hd3, hcbT, hcb, hcb2]
    outs = pl.pallas_call(
        _body,
        grid=(n,),
        in_specs=[item(36), item(360)]
                 + [full(a) for a in tws] + [full(a) for a in hws],
        out_specs=[item(9)] * 4 + [item(90)] * 4,
        out_shape=[oph(9)] * 4 + [oph(90)] * 4,
        compiler_params=pltpu.CompilerParams(
            dimension_semantics=("parallel",)),
    )(tin4, hin4, *tws, *hws)

    tout = jnp.stack(outs[0:4], axis=2).reshape(n, _T, 9)
    hout = jnp.stack(outs[4:8], axis=2).reshape(n, _T, 90)
    xo = jnp.concatenate([tout[..., :6], hout, tout[..., 6:]], axis=-1)
    x_out = jnp.concatenate([xo[:Bs], xo[Bs:]], axis=-1)
    return (x_out, jnp.array([1e30], jnp.float32),
            jnp.array([1e30], jnp.float32))


# speedup vs baseline: 1.3184x; 1.2264x over previous
import jax
import jax.numpy as jnp
from jax import lax
from jax.experimental import pallas as pl
from jax.experimental.pallas import tpu as pltpu

_B, _T, _NF = 16, 1024, 198
_CD = 256
_NCB = 512
_SH = _NF // 2
_TQ = _T // 4


def _sd(x):
    return jnp.concatenate([jnp.zeros_like(x[:1]), x[:-1]], axis=0)


def _su(x):
    return jnp.concatenate([x[1:], jnp.zeros_like(x[:1])], axis=0)


def _mm(a, b):
    return jnp.dot(a, b, preferred_element_type=jnp.float32)


def _act(x):
    return jnp.maximum(x, 0.0).astype(jnp.bfloat16)


def _hl(a, w, n=_CD):
    r = jnp.dot(a, w, preferred_element_type=jnp.float32)
    return r[:, :n] + r[:, n:2 * n]


def _hl90(a, w):
    r = jnp.dot(a, w, preferred_element_type=jnp.float32)
    return r[:, 0:90] + r[:, 128:218]


def _path(xp, w0, wl1, wl2, wl3, wd0, wd1, wd2, wd3, cbT, cb, cb2, hp):
    cat = lambda *ps: jnp.concatenate(ps, axis=1)
    R = range(len(xp))

    xs = [_sd(x) for x in xp]
    xu = [_su(x) for x in xp]
    if hp:
        h0 = [_act(_hl(xs[k], w0[0]) + _hl(xp[k], w0[1])) for k in R]
        h1 = [_act(_hl(xp[k], w0[2])) for k in R]
        h2 = [_act(_hl(xp[k], w0[3])) for k in R]
        h3 = [_act(_hl(xp[k], w0[4]) + _hl(xu[k], w0[5])) for k in R]
    else:
        h0 = [_act(_mm(xs[k], w0[0]) + _mm(xp[k], w0[1])) for k in R]
        h1 = [_act(_mm(xp[k], w0[2])) for k in R]
        h2 = [_act(_mm(xp[k], w0[3])) for k in R]
        h3 = [_act(_mm(xp[k], w0[4]) + _mm(xu[k], w0[5])) for k in R]
    ye = [_act(_mm(cat(_sd(h3[k]), h0[k], h1[k], h2[k]), wl1)) for k in R]
    yo = [_act(_mm(cat(h1[k], h2[k], h3[k], _su(h0[k])), wl1)) for k in R]
    h = [_act(_mm(cat(_sd(yo[k]), ye[k], yo[k], _su(ye[k])), wl2))
         for k in R]
    hs = [_sd(x) for x in h]
    hu = [_su(x) for x in h]
    z = [_hl(hs[k], wl3[0]) + _hl(h[k], wl3[1]) + _hl(hu[k], wl3[2])
         for k in R]

    zb = [x.astype(jnp.bfloat16) for x in z]
    zz = [jnp.sum(x * x, axis=-1, keepdims=True) for x in z]
    zc = [_hl(x, cbT, n=_NCB) for x in zb]
    dd = [zz[k] - 2.0 * zc[k] + cb2 for k in R]
    idx = [jnp.argmin(x, axis=-1)[:, None] for x in dd]
    iota = lax.broadcasted_iota(jnp.int32, (_TQ, _NCB), 1)
    oh = [(iota == i).astype(jnp.bfloat16) for i in idx]
    q = [_mm(x, cb).astype(jnp.bfloat16) for x in oh]

    qs = [_sd(x) for x in q]
    qu = [_su(x) for x in q]
    h = [_act(_hl(qs[k], wd0[0]) + _hl(q[k], wd0[1]) + _hl(qu[k], wd0[2]))
         for k in R]
    hs = [_sd(x) for x in h]
    hu = [_su(x) for x in h]
    ge = [_act(_hl(hs[k], wd1[0]) + _hl(h[k], wd1[1]) + _hl(h[k], wd1[2]))
          for k in R]
    go = [_act(_hl(h[k], wd1[0]) + _hl(h[k], wd1[1]) + _hl(hu[k], wd1[2]))
          for k in R]
    gos = [_sd(x) for x in go]
    geu = [_su(x) for x in ge]
    o0 = [_act(_hl(gos[k], wd2[0]) + _hl(ge[k], wd2[1])
               + _hl(ge[k], wd2[2])) for k in R]
    o1 = [_act(_hl(ge[k], wd2[0]) + _hl(ge[k], wd2[1])
               + _hl(go[k], wd2[2])) for k in R]
    o2 = [_act(_hl(ge[k], wd2[0]) + _hl(go[k], wd2[1])
               + _hl(go[k], wd2[2])) for k in R]
    o3 = [_act(_hl(go[k], wd2[0]) + _hl(go[k], wd2[1])
               + _hl(geu[k], wd2[2])) for k in R]

    o3s = [_sd(x) for x in o3]
    o0u = [_su(x) for x in o0]
    if hp:
        y0 = [_hl90(o3s[k], wd3[0]) + _hl90(o0[k], wd3[1])
              + _hl90(o1[k], wd3[2]) for k in R]
        y1 = [_hl90(o0[k], wd3[0]) + _hl90(o1[k], wd3[1])
              + _hl90(o2[k], wd3[2]) for k in R]
        y2 = [_hl90(o1[k], wd3[0]) + _hl90(o2[k], wd3[1])
              + _hl90(o3[k], wd3[2]) for k in R]
        y3 = [_hl90(o2[k], wd3[0]) + _hl90(o3[k], wd3[1])
              + _hl90(o0u[k], wd3[2]) for k in R]
    else:
        y0 = [_mm(cat(o3s[k], o0[k], o1[k]), wd3) for k in R]
        y1 = [_mm(cat(o0[k], o1[k], o2[k]), wd3) for k in R]
        y2 = [_mm(cat(o1[k], o2[k], o3[k]), wd3) for k in R]
        y3 = [_mm(cat(o2[k], o3[k], o0u[k]), wd3) for k in R]
    return y0, y1, y2, y3


def _body(tin, hin,
          tw0, tl1, tl2, tl3, td0, td1, td2, td3, tcbT, tcb, tcb2,
          hw0, hl1, hl2, hl3, hd0, hd1, hd2, hd3, hcbT, hcb, hcb2,
          t0, t1, t2, t3, p0, p1, p2, p3):
    ty = _path([tin[0], tin[1]], tw0, tl1[...], tl2[...], tl3[...],
               td0[...], td1[...], td2[...], td3[...], tcbT[...], tcb[...],
               tcb2[...], False)
    for r, ys in zip((t0, t1, t2, t3), ty):
        r[0], r[1] = ys
    hy = _path([hin[0], hin[1]], hw0, hl1[...], hl2[...], hl3[...],
               hd0[...], hd1[...], hd2[...], hd3[...], hcbT[...], hcb[...],
               hcb2[...], True)
    for r, ys in zip((p0, p1, p2, p3), hy):
        r[0], r[1] = ys


def _taps(w):
    return [w[:, :, k].T for k in range(w.shape[2])]


def _lo(m):
    return (m - m.astype(jnp.bfloat16).astype(jnp.float32)).astype(
        jnp.bfloat16).astype(jnp.float32)


def _kcat(mats):
    return jnp.concatenate(mats, axis=0).astype(jnp.bfloat16)


def _npair(m, pad=0):
    z = jnp.zeros((m.shape[0], pad), m.dtype)
    return jnp.concatenate([m, z, _lo(m), z], axis=1).astype(jnp.bfloat16)


def _nhilo(w, pad=0):
    return jnp.stack([_npair(m, pad) for m in _taps(w)])


def _pack_l0(w, with_lo):
    t0, t1, t2 = _taps(w)
    z = jnp.zeros_like(t0)

    def blk(b0, b1, b2, b3):
        return jnp.concatenate([b0, b1, b2, b3], axis=0)

    mats = [
        blk(z, z, z, t0),
        blk(t1, t2, z, z),
        blk(t0, t1, t2, z),
        blk(z, t0, t1, t2),
        blk(z, z, t0, t1),
        blk(t2, z, z, z),
    ]
    if with_lo:
        mats = [jnp.concatenate([m, _lo(m)], axis=1) for m in mats]
    return jnp.stack(mats).astype(jnp.bfloat16)


def kernel(features, traj_enc_w0, traj_enc_w1, traj_enc_w2, traj_enc_w3,
           traj_codebook, traj_dec_w0, traj_dec_w1, traj_dec_w2, traj_dec_w3,
           hp_enc_w0, hp_enc_w1, hp_enc_w2, hp_enc_w3, hp_codebook,
           hp_dec_w0, hp_dec_w1, hp_dec_w2, hp_dec_w3):
    Bs = features.shape[0]
    n = 2 * Bs
    x = jnp.concatenate([features[:, :, :_SH], features[:, :, _SH:]], axis=0)
    tin = jnp.concatenate([x[..., :6], x[..., _SH - 3:]], axis=-1)
    hin = x[..., 6:_SH - 3]
    tin4 = tin.reshape(n, _TQ, 4 * 9).astype(jnp.bfloat16)
    hin4 = hin.reshape(n, _TQ, 4 * 90).astype(jnp.bfloat16)

    tw0 = _pack_l0(traj_enc_w0, False)
    hw0 = _pack_l0(hp_enc_w0, True)
    tl1, tl2 = _kcat(_taps(traj_enc_w1)), _kcat(_taps(traj_enc_w2))
    hl1, hl2 = _kcat(_taps(hp_enc_w1)), _kcat(_taps(hp_enc_w2))
    tl3, td0 = _nhilo(traj_enc_w3), _nhilo(traj_dec_w0)
    td1, td2 = _nhilo(traj_dec_w1), _nhilo(traj_dec_w2)
    hl3, hd0 = _nhilo(hp_enc_w3), _nhilo(hp_dec_w0)
    hd1, hd2 = _nhilo(hp_dec_w1), _nhilo(hp_dec_w2)
    td3 = _kcat(_taps(traj_dec_w3))
    hd3 = _nhilo(hp_dec_w3, pad=38)
    tcbT = jnp.concatenate([traj_codebook.T, _lo(traj_codebook.T)],
                           axis=1).astype(jnp.bfloat16)
    hcbT = jnp.concatenate([hp_codebook.T, _lo(hp_codebook.T)],
                           axis=1).astype(jnp.bfloat16)
    tcb = traj_codebook.astype(jnp.bfloat16)
    hcb = hp_codebook.astype(jnp.bfloat16)
    tcb2 = jnp.sum(traj_codebook * traj_codebook, -1)[None]
    hcb2 = jnp.sum(hp_codebook * hp_codebook, -1)[None]

    full = lambda a: pl.BlockSpec(a.shape, lambda i: (0,) * a.ndim)
    item = lambda c: pl.BlockSpec((2, _TQ, c), lambda i: (i, 0, 0))
    oph = lambda c: jax.ShapeDtypeStruct((n, _TQ, c), jnp.float32)

    tws = [tw0, tl1, tl2, tl3, td0, td1, td2, td3, tcbT, tcb, tcb2]
    hws = [hw0, hl1, hl2, hl3, hd0, hd1, hd2, hd3, hcbT, hcb, hcb2]
    outs = pl.pallas_call(
        _body,
        grid=(n // 2,),
        in_specs=[item(36), item(360)]
                 + [full(a) for a in tws] + [full(a) for a in hws],
        out_specs=[item(9)] * 4 + [item(90)] * 4,
        out_shape=[oph(9)] * 4 + [oph(90)] * 4,
        compiler_params=pltpu.CompilerParams(
            dimension_semantics=("parallel",)),
    )(tin4, hin4, *tws, *hws)

    tout = jnp.stack(outs[0:4], axis=2).reshape(n, _T, 9)
    hout = jnp.stack(outs[4:8], axis=2).reshape(n, _T, 90)
    xo = jnp.concatenate([tout[..., :6], hout, tout[..., 6:]], axis=-1)
    x_out = jnp.concatenate([xo[:Bs], xo[Bs:]], axis=-1)
    return (x_out, jnp.array([1e30], jnp.float32),
            jnp.array([1e30], jnp.float32))


# speedup vs baseline: 1.3469x; 1.0216x over previous
import jax
import jax.numpy as jnp
from jax import lax
from jax.experimental import pallas as pl
from jax.experimental.pallas import tpu as pltpu

_B, _T, _NF = 16, 1024, 198
_CD = 256
_NCB = 512
_SH = _NF // 2
_TQ = _T // 4


def _sd(x):
    return jnp.concatenate([jnp.zeros_like(x[:1]), x[:-1]], axis=0)


def _su(x):
    return jnp.concatenate([x[1:], jnp.zeros_like(x[:1])], axis=0)


def _mm(a, b):
    return jnp.dot(a, b, preferred_element_type=jnp.float32)


def _act(x):
    return jnp.maximum(x, 0.0).astype(jnp.bfloat16)


def _hl(a, w, n=_CD):
    r = jnp.dot(a, w, preferred_element_type=jnp.float32)
    return r[:, :n] + r[:, n:2 * n]


def _hl90(a, w):
    r = jnp.dot(a, w, preferred_element_type=jnp.float32)
    return r[:, 0:90] + r[:, 128:218]


def _path(xp, w0, wl1, wl2, wl3, wd0, wd1, wd2, wd3, cbT, cb, cb2, hp):
    cat = lambda *ps: jnp.concatenate(ps, axis=1)
    R = range(len(xp))

    xs = [_sd(x) for x in xp]
    xu = [_su(x) for x in xp]
    if hp:
        h0 = [_act(_hl(xs[k], w0[0]) + _hl(xp[k], w0[1])) for k in R]
        h1 = [_act(_hl(xp[k], w0[2])) for k in R]
        h2 = [_act(_hl(xp[k], w0[3])) for k in R]
        h3 = [_act(_hl(xp[k], w0[4]) + _hl(xu[k], w0[5])) for k in R]
    else:
        h0 = [_act(_mm(xs[k], w0[0]) + _mm(xp[k], w0[1])) for k in R]
        h1 = [_act(_mm(xp[k], w0[2])) for k in R]
        h2 = [_act(_mm(xp[k], w0[3])) for k in R]
        h3 = [_act(_mm(xp[k], w0[4]) + _mm(xu[k], w0[5])) for k in R]
    ye = [_act(_mm(cat(_sd(h3[k]), h0[k], h1[k], h2[k]), wl1)) for k in R]
    yo = [_act(_mm(cat(h1[k], h2[k], h3[k], _su(h0[k])), wl1)) for k in R]
    h = [_act(_mm(cat(_sd(yo[k]), ye[k], yo[k], _su(ye[k])), wl2))
         for k in R]
    hs = [_sd(x) for x in h]
    hu = [_su(x) for x in h]
    z = [_hl(hs[k], wl3[0]) + _hl(h[k], wl3[1]) + _hl(hu[k], wl3[2])
         for k in R]

    zb = [x.astype(jnp.bfloat16) for x in z]
    zz = [jnp.sum(x * x, axis=-1, keepdims=True) for x in z]
    zc = [_hl(x, cbT, n=_NCB) for x in zb]
    dd = [zz[k] - 2.0 * zc[k] + cb2 for k in R]
    idx = [jnp.argmin(x, axis=-1)[:, None] for x in dd]
    iota = lax.broadcasted_iota(jnp.int32, (_TQ, _NCB), 1)
    oh = [(iota == i).astype(jnp.bfloat16) for i in idx]
    q = [_mm(x, cb).astype(jnp.bfloat16) for x in oh]

    qs = [_sd(x) for x in q]
    qu = [_su(x) for x in q]
    h = [_act(_hl(qs[k], wd0[0]) + _hl(q[k], wd0[1]) + _hl(qu[k], wd0[2]))
         for k in R]
    hs = [_sd(x) for x in h]
    hu = [_su(x) for x in h]
    ge = [_act(_hl(hs[k], wd1[0]) + _hl(h[k], wd1[1]) + _hl(h[k], wd1[2]))
          for k in R]
    go = [_act(_hl(h[k], wd1[0]) + _hl(h[k], wd1[1]) + _hl(hu[k], wd1[2]))
          for k in R]
    gos = [_sd(x) for x in go]
    geu = [_su(x) for x in ge]
    o0 = [_act(_hl(gos[k], wd2[0]) + _hl(ge[k], wd2[1])
               + _hl(ge[k], wd2[2])) for k in R]
    o1 = [_act(_hl(ge[k], wd2[0]) + _hl(ge[k], wd2[1])
               + _hl(go[k], wd2[2])) for k in R]
    o2 = [_act(_hl(ge[k], wd2[0]) + _hl(go[k], wd2[1])
               + _hl(go[k], wd2[2])) for k in R]
    o3 = [_act(_hl(go[k], wd2[0]) + _hl(go[k], wd2[1])
               + _hl(geu[k], wd2[2])) for k in R]

    o3s = [_sd(x) for x in o3]
    o0u = [_su(x) for x in o0]
    if hp:
        y0 = [_hl90(o3s[k], wd3[0]) + _hl90(o0[k], wd3[1])
              + _hl90(o1[k], wd3[2]) for k in R]
        y1 = [_hl90(o0[k], wd3[0]) + _hl90(o1[k], wd3[1])
              + _hl90(o2[k], wd3[2]) for k in R]
        y2 = [_hl90(o1[k], wd3[0]) + _hl90(o2[k], wd3[1])
              + _hl90(o3[k], wd3[2]) for k in R]
        y3 = [_hl90(o2[k], wd3[0]) + _hl90(o3[k], wd3[1])
              + _hl90(o0u[k], wd3[2]) for k in R]
    else:
        y0 = [_mm(cat(o3s[k], o0[k], o1[k]), wd3) for k in R]
        y1 = [_mm(cat(o0[k], o1[k], o2[k]), wd3) for k in R]
        y2 = [_mm(cat(o1[k], o2[k], o3[k]), wd3) for k in R]
        y3 = [_mm(cat(o2[k], o3[k], o0u[k]), wd3) for k in R]
    return y0, y1, y2, y3


def _body(tin, hin,
          tw0, tl1, tl2, tl3, td0, td1, td2, td3, tcbT, tcb, tcb2,
          hw0, hl1, hl2, hl3, hd0, hd1, hd2, hd3, hcbT, hcb, hcb2,
          t0, t1, t2, t3, p0, p1, p2, p3):
    ty = _path([tin[0], tin[1], tin[2], tin[3]], tw0, tl1[...], tl2[...],
               tl3[...], td0[...], td1[...], td2[...], td3[...], tcbT[...],
               tcb[...], tcb2[...], False)
    for r, ys in zip((t0, t1, t2, t3), ty):
        r[0], r[1], r[2], r[3] = ys
    hy = _path([hin[0], hin[1], hin[2], hin[3]], hw0, hl1[...], hl2[...],
               hl3[...], hd0[...], hd1[...], hd2[...], hd3[...], hcbT[...],
               hcb[...], hcb2[...], True)
    for r, ys in zip((p0, p1, p2, p3), hy):
        r[0], r[1], r[2], r[3] = ys


def _taps(w):
    return [w[:, :, k].T for k in range(w.shape[2])]


def _lo(m):
    return (m - m.astype(jnp.bfloat16).astype(jnp.float32)).astype(
        jnp.bfloat16).astype(jnp.float32)


def _kcat(mats):
    return jnp.concatenate(mats, axis=0).astype(jnp.bfloat16)


def _npair(m, pad=0):
    z = jnp.zeros((m.shape[0], pad), m.dtype)
    return jnp.concatenate([m, z, _lo(m), z], axis=1).astype(jnp.bfloat16)


def _nhilo(w, pad=0):
    return jnp.stack([_npair(m, pad) for m in _taps(w)])


def _pack_l0(w, with_lo):
    t0, t1, t2 = _taps(w)
    z = jnp.zeros_like(t0)

    def blk(b0, b1, b2, b3):
        return jnp.concatenate([b0, b1, b2, b3], axis=0)

    mats = [
        blk(z, z, z, t0),
        blk(t1, t2, z, z),
        blk(t0, t1, t2, z),
        blk(z, t0, t1, t2),
        blk(z, z, t0, t1),
        blk(t2, z, z, z),
    ]
    if with_lo:
        mats = [jnp.concatenate([m, _lo(m)], axis=1) for m in mats]
    return jnp.stack(mats).astype(jnp.bfloat16)


def kernel(features, traj_enc_w0, traj_enc_w1, traj_enc_w2, traj_enc_w3,
           traj_codebook, traj_dec_w0, traj_dec_w1, traj_dec_w2, traj_dec_w3,
           hp_enc_w0, hp_enc_w1, hp_enc_w2, hp_enc_w3, hp_codebook,
           hp_dec_w0, hp_dec_w1, hp_dec_w2, hp_dec_w3):
    Bs = features.shape[0]
    n = 2 * Bs
    x = jnp.concatenate([features[:, :, :_SH], features[:, :, _SH:]], axis=0)
    tin = jnp.concatenate([x[..., :6], x[..., _SH - 3:]], axis=-1)
    hin = x[..., 6:_SH - 3]
    tin4 = tin.reshape(n, _TQ, 4 * 9).astype(jnp.bfloat16)
    hin4 = hin.reshape(n, _TQ, 4 * 90).astype(jnp.bfloat16)

    tw0 = _pack_l0(traj_enc_w0, False)
    hw0 = _pack_l0(hp_enc_w0, True)
    tl1, tl2 = _kcat(_taps(traj_enc_w1)), _kcat(_taps(traj_enc_w2))
    hl1, hl2 = _kcat(_taps(hp_enc_w1)), _kcat(_taps(hp_enc_w2))
    tl3, td0 = _nhilo(traj_enc_w3), _nhilo(traj_dec_w0)
    td1, td2 = _nhilo(traj_dec_w1), _nhilo(traj_dec_w2)
    hl3, hd0 = _nhilo(hp_enc_w3), _nhilo(hp_dec_w0)
    hd1, hd2 = _nhilo(hp_dec_w1), _nhilo(hp_dec_w2)
    td3 = _kcat(_taps(traj_dec_w3))
    hd3 = _nhilo(hp_dec_w3, pad=38)
    tcbT = jnp.concatenate([traj_codebook.T, _lo(traj_codebook.T)],
                           axis=1).astype(jnp.bfloat16)
    hcbT = jnp.concatenate([hp_codebook.T, _lo(hp_codebook.T)],
                           axis=1).astype(jnp.bfloat16)
    tcb = traj_codebook.astype(jnp.bfloat16)
    hcb = hp_codebook.astype(jnp.bfloat16)
    tcb2 = jnp.sum(traj_codebook * traj_codebook, -1)[None]
    hcb2 = jnp.sum(hp_codebook * hp_codebook, -1)[None]

    full = lambda a: pl.BlockSpec(a.shape, lambda i: (0,) * a.ndim)
    item = lambda c: pl.BlockSpec((4, _TQ, c), lambda i: (i, 0, 0))
    oph = lambda c: jax.ShapeDtypeStruct((n, _TQ, c), jnp.float32)

    tws = [tw0, tl1, tl2, tl3, td0, td1, td2, td3, tcbT, tcb, tcb2]
    hws = [hw0, hl1, hl2, hl3, hd0, hd1, hd2, hd3, hcbT, hcb, hcb2]
    outs = pl.pallas_call(
        _body,
        grid=(n // 4,),
        in_specs=[item(36), item(360)]
                 + [full(a) for a in tws] + [full(a) for a in hws],
        out_specs=[item(9)] * 4 + [item(90)] * 4,
        out_shape=[oph(9)] * 4 + [oph(90)] * 4,
        compiler_params=pltpu.CompilerParams(
            dimension_semantics=("parallel",)),
    )(tin4, hin4, *tws, *hws)

    tout = jnp.stack(outs[0:4], axis=2).reshape(n, _T, 9)
    hout = jnp.stack(outs[4:8], axis=2).reshape(n, _T, 90)
    xo = jnp.concatenate([tout[..., :6], hout, tout[..., 6:]], axis=-1)
    x_out = jnp.concatenate([xo[:Bs], xo[Bs:]], axis=-1)
    return (x_out, jnp.array([1e30], jnp.float32),
            jnp.array([1e30], jnp.float32))
